# baseline (speedup 1.0000x reference)
import functools

import jax
import jax.numpy as jnp
from jax import lax
from jax.experimental import pallas as pl
from jax.experimental.pallas import tpu as pltpu
from jax.experimental.pallas import tpu_sc as plsc

_N = 10000
_NPAD = 10112
_RPT = _NPAD // 16
_E = 160000
_CORES = 2
_TILES = 16
_EPT = 5120
_EPAD = _CORES * _TILES * _EPT
_GAMMA = 0.5
_F32 = jnp.float32


def _sc_edge_scatter(table, gidx, sidx, width, g, nbuf):
    chunks = _EPT // g
    assert _EPT % g == 0 and chunks % nbuf == 0
    gidx = gidx.reshape(_CORES, _TILES, chunks, g)
    sidx = sidx.reshape(_CORES, _TILES, chunks, g)
    mesh = plsc.VectorSubcoreMesh(core_axis_name="c", subcore_axis_name="s")
    zeros = jnp.zeros((_RPT, width), _F32)

    def body(table_h, gidx_h, sidx_h, zeros_h, out_h, gi_v, si_v,
             gbufs, acc_s, sems_g, sems_s):
        c = lax.axis_index("c")
        s = lax.axis_index("s")
        r0 = s * _RPT
        pltpu.sync_copy(zeros_h, acc_s.at[pl.ds(r0, _RPT)])
        pltpu.sync_copy(gidx_h.at[c, s], gi_v)
        pltpu.sync_copy(sidx_h.at[c, s], si_v)
        plsc.subcore_barrier()

        for b in range(nbuf):
            pltpu.async_copy(table_h.at[gi_v.at[b]], gbufs[b], sems_g[b])

        last = chunks - 1

        @pl.loop(0, chunks, step=nbuf)
        def _chunk(j):
            for b in range(nbuf):
                pltpu.make_async_copy(table_h.at[gi_v.at[0]], gbufs[b],
                                      sems_g[b]).wait()
                pltpu.async_copy(gbufs[b], acc_s.at[si_v.at[j + b]],
                                 sems_s[b], add=True)
                if b >= 1:
                    pltpu.make_async_copy(gbufs[b - 1], acc_s.at[si_v.at[0]],
                                          sems_s[b - 1]).wait()
                    pltpu.async_copy(
                        table_h.at[gi_v.at[jnp.minimum(j + nbuf + b - 1,
                                                       last)]],
                        gbufs[b - 1], sems_g[b - 1])
            pltpu.make_async_copy(gbufs[nbuf - 1], acc_s.at[si_v.at[0]],
                                  sems_s[nbuf - 1]).wait()
            pltpu.async_copy(
                table_h.at[gi_v.at[jnp.minimum(j + 2 * nbuf - 1, last)]],
                gbufs[nbuf - 1], sems_g[nbuf - 1])

        for b in range(nbuf):
            pltpu.make_async_copy(table_h.at[gi_v.at[0]], gbufs[b],
                                  sems_g[b]).wait()
        plsc.subcore_barrier()
        pltpu.sync_copy(acc_s.at[pl.ds(r0, _RPT)],
                        out_h.at[c, pl.ds(r0, _RPT)])

    fn = pl.kernel(
        body,
        out_type=jax.ShapeDtypeStruct((_CORES, _NPAD, width), _F32),
        mesh=mesh,
        scratch_types=[
            pltpu.VMEM((chunks, g), jnp.int32),
            pltpu.VMEM((chunks, g), jnp.int32),
            [pltpu.VMEM((g, width), _F32) for _ in range(nbuf)],
            pltpu.VMEM_SHARED((_NPAD, width), _F32),
            [pltpu.SemaphoreType.DMA for _ in range(nbuf)],
            [pltpu.SemaphoreType.DMA for _ in range(nbuf)],
        ],
        compiler_params=pltpu.CompilerParams(use_tc_tiling_on_sc=False),
    )
    return fn(table, gidx, sidx, zeros)


_BM = 632


def _tc1_body(x_ref, w_ref, o_ref):
    i = pl.program_id(0)
    m = jnp.dot(x_ref[...], w_ref[...], preferred_element_type=_F32)
    row = i * _BM + lax.broadcasted_iota(jnp.int32, (_BM, 16), 0)
    col = lax.broadcasted_iota(jnp.int32, (_BM, 16), 1)
    ones = jnp.where((row < _N) & (col == 0), 1.0, 0.0).astype(_F32)
    o_ref[...] = jnp.concatenate([m, ones], axis=1)


def _tc_m0ext(x_pad, W_g0):
    return pl.pallas_call(
        _tc1_body,
        grid=(_NPAD // _BM,),
        in_specs=[pl.BlockSpec((_BM, 128), lambda i: (i, 0)),
                  pl.BlockSpec((128, 128), lambda i: (0, 0))],
        out_specs=pl.BlockSpec((_BM, 144), lambda i: (i, 0)),
        out_shape=jax.ShapeDtypeStruct((_NPAD, 144), _F32),
    )(x_pad, W_g0)


def _tc2_body(pA0, pA1, pB0, pB1, br1, wr2, br2, wf0, bf0, bg0, wg1,
              R_o, m1s_o, self_o, dinv_o):
    AW = pA0[0] + pA1[0]
    T1 = jnp.maximum(AW + br1[...], 0.0)
    R = jnp.maximum(
        jnp.dot(T1, wr2[...], preferred_element_type=_F32) + br2[...], 0.0)
    Rl = jnp.maximum(
        jnp.dot(R, wf0[...], preferred_element_type=_F32) + bf0[...], 0.0)
    S1e = pB0[0] + pB1[0]
    H = jnp.maximum(S1e[:, :128] + bg0[...], 0.0)
    Hm = H * jnp.exp(-_GAMMA * Rl)
    m1 = jnp.dot(Hm, wg1[...], preferred_element_type=_F32)
    deg = S1e[:, 128:129] + 1.0
    dinv = lax.rsqrt(deg)
    dinv64 = jnp.broadcast_to(dinv, (_BM, 64))
    R_o[...] = R
    m1s_o[...] = m1 * dinv64
    self_o[...] = m1 * dinv64 * dinv64
    dinv_o[...] = dinv64


def _tc_mid(pA, pB, b_r1, W_r2, b_r2, W_f0, b_f0, b_g0, W_g1):
    g = _NPAD // _BM
    row = lambda i: (i, 0)
    cst = lambda i: (0, 0)
    return pl.pallas_call(
        _tc2_body,
        grid=(g,),
        in_specs=[
            pl.BlockSpec((1, _BM, 128), lambda i: (0, i, 0)),
            pl.BlockSpec((1, _BM, 128), lambda i: (1, i, 0)),
            pl.BlockSpec((1, _BM, 144), lambda i: (0, i, 0)),
            pl.BlockSpec((1, _BM, 144), lambda i: (1, i, 0)),
            pl.BlockSpec((1, 128), cst), pl.BlockSpec((128, 128), cst),
            pl.BlockSpec((1, 128), cst), pl.BlockSpec((128, 128), cst),
            pl.BlockSpec((1, 128), cst), pl.BlockSpec((1, 128), cst),
            pl.BlockSpec((128, 64), cst),
        ],
        out_specs=[
            pl.BlockSpec((_BM, 128), row), pl.BlockSpec((_BM, 64), row),
            pl.BlockSpec((_BM, 64), row), pl.BlockSpec((_BM, 64), row),
        ],
        out_shape=[
            jax.ShapeDtypeStruct((_NPAD, 128), _F32),
            jax.ShapeDtypeStruct((_NPAD, 64), _F32),
            jax.ShapeDtypeStruct((_NPAD, 64), _F32),
            jax.ShapeDtypeStruct((_NPAD, 64), _F32),
        ],
    )(pA, pA, pB, pB, b_r1, W_r2, b_r2, W_f0, b_f0, b_g0, W_g1)


def _tc3_body(pC0, pC1, self_r, dinv_r, bg1, wd1, bd1, wd2, bd2, H2_o, X_o):
    S2 = pC0[0] + pC1[0]
    H2 = jnp.maximum(dinv_r[...] * S2 + self_r[...] + bg1[...], 0.0)
    T = jnp.maximum(
        jnp.dot(H2, wd1[...], preferred_element_type=_F32) + bd1[...], 0.0)
    X = jnp.maximum(
        jnp.dot(T, wd2[...], preferred_element_type=_F32) + bd2[...], 0.0)
    H2_o[...] = H2
    X_o[...] = X


def _tc_dec(pC, selfterm, dinv64, b_g1, W_d1, b_d1, W_d2, b_d2):
    g = _NPAD // _BM
    row = lambda i: (i, 0)
    cst = lambda i: (0, 0)
    return pl.pallas_call(
        _tc3_body,
        grid=(g,),
        in_specs=[
            pl.BlockSpec((1, _BM, 64), lambda i: (0, i, 0)),
            pl.BlockSpec((1, _BM, 64), lambda i: (1, i, 0)),
            pl.BlockSpec((_BM, 64), row), pl.BlockSpec((_BM, 64), row),
            pl.BlockSpec((1, 64), cst), pl.BlockSpec((64, 128), cst),
            pl.BlockSpec((1, 128), cst), pl.BlockSpec((128, 128), cst),
            pl.BlockSpec((1, 128), cst),
        ],
        out_specs=[pl.BlockSpec((_BM, 64), row), pl.BlockSpec((_BM, 128), row)],
        out_shape=[jax.ShapeDtypeStruct((_NPAD, 64), _F32),
                   jax.ShapeDtypeStruct((_NPAD, 128), _F32)],
    )(pC, pC, selfterm, dinv64, b_g1, W_d1, b_d1, W_d2, b_d2)


def _tc4_body(a_ref, b_ref, o_ref):
    o_ref[...] = lax.dot_general(
        a_ref[...], b_ref[...], (((1,), (1,)), ((), ())),
        preferred_element_type=_F32)


def _tc_ahat(H2):
    BM = 400
    return pl.pallas_call(
        _tc4_body,
        grid=(_N // BM,),
        in_specs=[pl.BlockSpec((BM, 64), lambda i: (i, 0)),
                  pl.BlockSpec((_N, 64), lambda i: (0, 0))],
        out_specs=pl.BlockSpec((BM, _N), lambda i: (i, 0)),
        out_shape=jax.ShapeDtypeStruct((_N, _N), _F32),
    )(H2, H2)


def kernel(x, W_r1, b_r1, W_r2, b_r2, W_f0, b_f0, W_g0, b_g0, W_g1, b_g1,
           W_d1, b_d1, W_d2, b_d2, edge_index):
    src = edge_index[0].astype(jnp.int32)
    dst = edge_index[1].astype(jnp.int32)
    pad = jnp.full((_EPAD - _E,), _N, jnp.int32)
    srcp = jnp.concatenate([src, pad])
    dstp = jnp.concatenate([dst, pad])

    xp = jnp.pad(x, ((0, _NPAD - _N), (0, 0)))
    W_r1p = jnp.pad(W_r1, ((0, _NPAD - _N), (0, 0)))

    r2 = lambda b: b.reshape(1, -1)

    m0ext = _tc_m0ext(xp, W_g0)
    pA = _sc_edge_scatter(W_r1p, dstp, srcp, 128, 128, 2)
    pB = _sc_edge_scatter(m0ext, srcp, dstp, 144, 80, 2)
    R, m1s, selfterm, dinv64 = _tc_mid(
        pA, pB, r2(b_r1), W_r2, r2(b_r2), W_f0, r2(b_f0), r2(b_g0), W_g1)
    pC = _sc_edge_scatter(m1s, srcp, dstp, 64, 128, 4)
    H2, X_hat = _tc_dec(pC, selfterm, dinv64, r2(b_g1), W_d1, r2(b_d1),
                        W_d2, r2(b_d2))
    A_hat = _tc_ahat(H2[:_N])
    return X_hat[:_N], A_hat, R[:_N]


# speedup vs baseline: 1.0213x; 1.0213x over previous
import functools

import jax
import jax.numpy as jnp
from jax import lax
from jax.experimental import pallas as pl
from jax.experimental.pallas import tpu as pltpu
from jax.experimental.pallas import tpu_sc as plsc

_N = 10000
_NPAD = 10112
_RPT = _NPAD // 16
_E = 160000
_CORES = 2
_TILES = 16
_EPT = 5120
_EPAD = _CORES * _TILES * _EPT
_GAMMA = 0.5
_F32 = jnp.float32


def _sc_edge_scatter(table, gidx, sidx, width, g, nbuf):
    chunks = _EPT // g
    assert _EPT % g == 0 and chunks % nbuf == 0
    gidx = gidx.reshape(_CORES, _TILES, chunks, g)
    sidx = sidx.reshape(_CORES, _TILES, chunks, g)
    mesh = plsc.VectorSubcoreMesh(core_axis_name="c", subcore_axis_name="s")
    zeros = jnp.zeros((_RPT, width), _F32)

    def body(table_h, gidx_h, sidx_h, zeros_h, out_h, gi_v, si_v,
             gbufs, acc_s, sems_g, sems_s):
        c = lax.axis_index("c")
        s = lax.axis_index("s")
        r0 = s * _RPT
        pltpu.sync_copy(zeros_h, acc_s.at[pl.ds(r0, _RPT)])
        pltpu.sync_copy(gidx_h.at[c, s], gi_v)
        pltpu.sync_copy(sidx_h.at[c, s], si_v)
        plsc.subcore_barrier()

        for b in range(nbuf):
            pltpu.async_copy(table_h.at[gi_v.at[b]], gbufs[b], sems_g[b])

        last = chunks - 1

        @pl.loop(0, chunks, step=nbuf)
        def _chunk(j):
            for b in range(nbuf):
                pltpu.make_async_copy(table_h.at[gi_v.at[0]], gbufs[b],
                                      sems_g[b]).wait()
                pltpu.async_copy(gbufs[b], acc_s.at[si_v.at[j + b]],
                                 sems_s[b], add=True).wait()
                pltpu.async_copy(
                    table_h.at[gi_v.at[jnp.minimum(j + nbuf + b, last)]],
                    gbufs[b], sems_g[b])

        for b in range(nbuf):
            pltpu.make_async_copy(table_h.at[gi_v.at[0]], gbufs[b],
                                  sems_g[b]).wait()
        plsc.subcore_barrier()
        pltpu.sync_copy(acc_s.at[pl.ds(r0, _RPT)],
                        out_h.at[c, pl.ds(r0, _RPT)])

    fn = pl.kernel(
        body,
        out_type=jax.ShapeDtypeStruct((_CORES, _NPAD, width), _F32),
        mesh=mesh,
        scratch_types=[
            pltpu.VMEM((chunks, g), jnp.int32),
            pltpu.VMEM((chunks, g), jnp.int32),
            [pltpu.VMEM((g, width), _F32) for _ in range(nbuf)],
            pltpu.VMEM_SHARED((_NPAD, width), _F32),
            [pltpu.SemaphoreType.DMA for _ in range(nbuf)],
            [pltpu.SemaphoreType.DMA for _ in range(nbuf)],
        ],
        compiler_params=pltpu.CompilerParams(use_tc_tiling_on_sc=False),
    )
    return fn(table, gidx, sidx, zeros)


_BM = 632


def _tc1_body(x_ref, w_ref, o_ref):
    i = pl.program_id(0)
    m = jnp.dot(x_ref[...], w_ref[...], preferred_element_type=_F32)
    row = i * _BM + lax.broadcasted_iota(jnp.int32, (_BM, 16), 0)
    col = lax.broadcasted_iota(jnp.int32, (_BM, 16), 1)
    ones = jnp.where((row < _N) & (col == 0), 1.0, 0.0).astype(_F32)
    o_ref[...] = jnp.concatenate([m, ones], axis=1)


def _tc_m0ext(x_pad, W_g0):
    return pl.pallas_call(
        _tc1_body,
        grid=(_NPAD // _BM,),
        in_specs=[pl.BlockSpec((_BM, 128), lambda i: (i, 0)),
                  pl.BlockSpec((128, 128), lambda i: (0, 0))],
        out_specs=pl.BlockSpec((_BM, 144), lambda i: (i, 0)),
        out_shape=jax.ShapeDtypeStruct((_NPAD, 144), _F32),
    )(x_pad, W_g0)


def _tc2_body(pA0, pA1, pB0, pB1, br1, wr2, br2, wf0, bf0, bg0, wg1,
              R_o, m1s_o, self_o, dinv_o):
    AW = pA0[0] + pA1[0]
    T1 = jnp.maximum(AW + br1[...], 0.0)
    R = jnp.maximum(
        jnp.dot(T1, wr2[...], preferred_element_type=_F32) + br2[...], 0.0)
    Rl = jnp.maximum(
        jnp.dot(R, wf0[...], preferred_element_type=_F32) + bf0[...], 0.0)
    S1e = pB0[0] + pB1[0]
    H = jnp.maximum(S1e[:, :128] + bg0[...], 0.0)
    Hm = H * jnp.exp(-_GAMMA * Rl)
    m1 = jnp.dot(Hm, wg1[...], preferred_element_type=_F32)
    deg = S1e[:, 128:129] + 1.0
    dinv = lax.rsqrt(deg)
    dinv64 = jnp.broadcast_to(dinv, (_BM, 64))
    R_o[...] = R
    m1s_o[...] = m1 * dinv64
    self_o[...] = m1 * dinv64 * dinv64
    dinv_o[...] = dinv64


def _tc_mid(pA, pB, b_r1, W_r2, b_r2, W_f0, b_f0, b_g0, W_g1):
    g = _NPAD // _BM
    row = lambda i: (i, 0)
    cst = lambda i: (0, 0)
    return pl.pallas_call(
        _tc2_body,
        grid=(g,),
        in_specs=[
            pl.BlockSpec((1, _BM, 128), lambda i: (0, i, 0)),
            pl.BlockSpec((1, _BM, 128), lambda i: (1, i, 0)),
            pl.BlockSpec((1, _BM, 144), lambda i: (0, i, 0)),
            pl.BlockSpec((1, _BM, 144), lambda i: (1, i, 0)),
            pl.BlockSpec((1, 128), cst), pl.BlockSpec((128, 128), cst),
            pl.BlockSpec((1, 128), cst), pl.BlockSpec((128, 128), cst),
            pl.BlockSpec((1, 128), cst), pl.BlockSpec((1, 128), cst),
            pl.BlockSpec((128, 64), cst),
        ],
        out_specs=[
            pl.BlockSpec((_BM, 128), row), pl.BlockSpec((_BM, 64), row),
            pl.BlockSpec((_BM, 64), row), pl.BlockSpec((_BM, 64), row),
        ],
        out_shape=[
            jax.ShapeDtypeStruct((_NPAD, 128), _F32),
            jax.ShapeDtypeStruct((_NPAD, 64), _F32),
            jax.ShapeDtypeStruct((_NPAD, 64), _F32),
            jax.ShapeDtypeStruct((_NPAD, 64), _F32),
        ],
    )(pA, pA, pB, pB, b_r1, W_r2, b_r2, W_f0, b_f0, b_g0, W_g1)


def _tc3_body(pC0, pC1, self_r, dinv_r, bg1, wd1, bd1, wd2, bd2, H2_o, X_o):
    S2 = pC0[0] + pC1[0]
    H2 = jnp.maximum(dinv_r[...] * S2 + self_r[...] + bg1[...], 0.0)
    T = jnp.maximum(
        jnp.dot(H2, wd1[...], preferred_element_type=_F32) + bd1[...], 0.0)
    X = jnp.maximum(
        jnp.dot(T, wd2[...], preferred_element_type=_F32) + bd2[...], 0.0)
    H2_o[...] = H2
    X_o[...] = X


def _tc_dec(pC, selfterm, dinv64, b_g1, W_d1, b_d1, W_d2, b_d2):
    g = _NPAD // _BM
    row = lambda i: (i, 0)
    cst = lambda i: (0, 0)
    return pl.pallas_call(
        _tc3_body,
        grid=(g,),
        in_specs=[
            pl.BlockSpec((1, _BM, 64), lambda i: (0, i, 0)),
            pl.BlockSpec((1, _BM, 64), lambda i: (1, i, 0)),
            pl.BlockSpec((_BM, 64), row), pl.BlockSpec((_BM, 64), row),
            pl.BlockSpec((1, 64), cst), pl.BlockSpec((64, 128), cst),
            pl.BlockSpec((1, 128), cst), pl.BlockSpec((128, 128), cst),
            pl.BlockSpec((1, 128), cst),
        ],
        out_specs=[pl.BlockSpec((_BM, 64), row), pl.BlockSpec((_BM, 128), row)],
        out_shape=[jax.ShapeDtypeStruct((_NPAD, 64), _F32),
                   jax.ShapeDtypeStruct((_NPAD, 128), _F32)],
    )(pC, pC, selfterm, dinv64, b_g1, W_d1, b_d1, W_d2, b_d2)


def _tc4_body(a_ref, b_ref, o_ref):
    o_ref[...] = lax.dot_general(
        a_ref[...], b_ref[...], (((1,), (1,)), ((), ())),
        preferred_element_type=_F32)


def _tc_ahat(H2):
    BM = 400
    return pl.pallas_call(
        _tc4_body,
        grid=(_N // BM,),
        in_specs=[pl.BlockSpec((BM, 64), lambda i: (i, 0)),
                  pl.BlockSpec((_N, 64), lambda i: (0, 0))],
        out_specs=pl.BlockSpec((BM, _N), lambda i: (i, 0)),
        out_shape=jax.ShapeDtypeStruct((_N, _N), _F32),
    )(H2, H2)


def kernel(x, W_r1, b_r1, W_r2, b_r2, W_f0, b_f0, W_g0, b_g0, W_g1, b_g1,
           W_d1, b_d1, W_d2, b_d2, edge_index):
    src = edge_index[0].astype(jnp.int32)
    dst = edge_index[1].astype(jnp.int32)
    pad = jnp.full((_EPAD - _E,), _N, jnp.int32)
    srcp = jnp.concatenate([src, pad])
    dstp = jnp.concatenate([dst, pad])

    xp = jnp.pad(x, ((0, _NPAD - _N), (0, 0)))
    W_r1p = jnp.pad(W_r1, ((0, _NPAD - _N), (0, 0)))

    r2 = lambda b: b.reshape(1, -1)

    m0ext = _tc_m0ext(xp, W_g0)
    pA = _sc_edge_scatter(W_r1p, dstp, srcp, 128, 128, 2)
    pB = _sc_edge_scatter(m0ext, srcp, dstp, 144, 80, 2)
    R, m1s, selfterm, dinv64 = _tc_mid(
        pA, pB, r2(b_r1), W_r2, r2(b_r2), W_f0, r2(b_f0), r2(b_g0), W_g1)
    pC = _sc_edge_scatter(m1s, srcp, dstp, 64, 128, 2)
    H2, X_hat = _tc_dec(pC, selfterm, dinv64, r2(b_g1), W_d1, r2(b_d1),
                        W_d2, r2(b_d2))
    A_hat = _tc_ahat(H2[:_N])
    return X_hat[:_N], A_hat, R[:_N]


# speedup vs baseline: 1.0885x; 1.0658x over previous
import functools

import jax
import jax.numpy as jnp
from jax import lax
from jax.experimental import pallas as pl
from jax.experimental.pallas import tpu as pltpu
from jax.experimental.pallas import tpu_sc as plsc

_N = 10000
_NPAD = 10112
_RPT = _NPAD // 16
_E = 160000
_CORES = 2
_TILES = 16
_EPT = 5120
_EPAD = _CORES * _TILES * _EPT
_GAMMA = 0.5
_F32 = jnp.float32


def _sc_edge_scatter(table, gidx, sidx, width, g, nbuf):
    chunks = _EPT // g
    assert _EPT % g == 0 and chunks % nbuf == 0
    gidx = gidx.reshape(_CORES, _TILES, chunks, g)
    sidx = sidx.reshape(_CORES, _TILES, chunks, g)
    mesh = plsc.VectorSubcoreMesh(core_axis_name="c", subcore_axis_name="s")
    zeros = jnp.zeros((_RPT, width), _F32)

    def body(table_h, gidx_h, sidx_h, zeros_h, out_h, gi_v, si_v,
             gbufs, acc_s, sems_g, sems_s):
        c = lax.axis_index("c")
        s = lax.axis_index("s")
        r0 = s * _RPT
        pltpu.sync_copy(zeros_h, acc_s.at[pl.ds(r0, _RPT)])
        pltpu.sync_copy(gidx_h.at[c, s], gi_v)
        pltpu.sync_copy(sidx_h.at[c, s], si_v)
        plsc.subcore_barrier()

        for b in range(nbuf):
            pltpu.async_copy(table_h.at[gi_v.at[b]], gbufs[b], sems_g[b])

        last = chunks - 1

        @pl.loop(0, chunks, step=nbuf)
        def _chunk(j):
            for b in range(nbuf):
                pltpu.make_async_copy(table_h.at[gi_v.at[0]], gbufs[b],
                                      sems_g[b]).wait()
                pltpu.async_copy(gbufs[b], acc_s.at[si_v.at[j + b]],
                                 sems_s[b], add=True).wait()
                pltpu.async_copy(
                    table_h.at[gi_v.at[jnp.minimum(j + nbuf + b, last)]],
                    gbufs[b], sems_g[b])

        for b in range(nbuf):
            pltpu.make_async_copy(table_h.at[gi_v.at[0]], gbufs[b],
                                  sems_g[b]).wait()
        plsc.subcore_barrier()
        pltpu.sync_copy(acc_s.at[pl.ds(r0, _RPT)],
                        out_h.at[c, pl.ds(r0, _RPT)])

    fn = pl.kernel(
        body,
        out_type=jax.ShapeDtypeStruct((_CORES, _NPAD, width), _F32),
        mesh=mesh,
        scratch_types=[
            pltpu.VMEM((chunks, g), jnp.int32),
            pltpu.VMEM((chunks, g), jnp.int32),
            [pltpu.VMEM((g, width), _F32) for _ in range(nbuf)],
            pltpu.VMEM_SHARED((_NPAD, width), _F32),
            [pltpu.SemaphoreType.DMA for _ in range(nbuf)],
            [pltpu.SemaphoreType.DMA for _ in range(nbuf)],
        ],
        compiler_params=pltpu.CompilerParams(use_tc_tiling_on_sc=False),
    )
    return fn(table, gidx, sidx, zeros)


_BM = 632


def _tc1_body(x_ref, w_ref, o_ref):
    i = pl.program_id(0)
    m = jnp.dot(x_ref[...], w_ref[...], preferred_element_type=_F32)
    row = i * _BM + lax.broadcasted_iota(jnp.int32, (_BM, 16), 0)
    col = lax.broadcasted_iota(jnp.int32, (_BM, 16), 1)
    ones = jnp.where((row < _N) & (col == 0), 1.0, 0.0).astype(_F32)
    o_ref[...] = jnp.concatenate([m, ones], axis=1)


def _tc_m0ext(x_pad, W_g0):
    return pl.pallas_call(
        _tc1_body,
        grid=(_NPAD // _BM,),
        in_specs=[pl.BlockSpec((_BM, 128), lambda i: (i, 0)),
                  pl.BlockSpec((128, 128), lambda i: (0, 0))],
        out_specs=pl.BlockSpec((_BM, 144), lambda i: (i, 0)),
        out_shape=jax.ShapeDtypeStruct((_NPAD, 144), _F32),
    )(x_pad, W_g0)


def _tc2_body(pA0, pA1, pB0, pB1, br1, wr2, br2, wf0, bf0, bg0, wg1,
              R_o, m1s_o, self_o, dinv_o):
    AW = pA0[...] + pA1[...]
    T1 = jnp.maximum(AW + br1[...], 0.0)
    R = jnp.maximum(
        jnp.dot(T1, wr2[...], preferred_element_type=_F32) + br2[...], 0.0)
    Rl = jnp.maximum(
        jnp.dot(R, wf0[...], preferred_element_type=_F32) + bf0[...], 0.0)
    S1e = pB0[...] + pB1[...]
    H = jnp.maximum(S1e[:, :128] + bg0[...], 0.0)
    Hm = H * jnp.exp(-_GAMMA * Rl)
    m1 = jnp.dot(Hm, wg1[...], preferred_element_type=_F32)
    deg = S1e[:, 128:129] + 1.0
    dinv = lax.rsqrt(deg)
    dinv64 = jnp.broadcast_to(dinv, (_BM, 64))
    R_o[...] = R
    m1s_o[...] = m1 * dinv64
    self_o[...] = m1 * dinv64 * dinv64
    dinv_o[...] = dinv64


def _tc_mid(pA, pB, b_r1, W_r2, b_r2, W_f0, b_f0, b_g0, W_g1):
    g = _NPAD // _BM
    row = lambda i: (i, 0)
    cst = lambda i: (0, 0)
    return pl.pallas_call(
        _tc2_body,
        grid=(g,),
        in_specs=[
            pl.BlockSpec((_BM, 128), row), pl.BlockSpec((_BM, 128), row),
            pl.BlockSpec((_BM, 144), row), pl.BlockSpec((_BM, 144), row),
            pl.BlockSpec((1, 128), cst), pl.BlockSpec((128, 128), cst),
            pl.BlockSpec((1, 128), cst), pl.BlockSpec((128, 128), cst),
            pl.BlockSpec((1, 128), cst), pl.BlockSpec((1, 128), cst),
            pl.BlockSpec((128, 64), cst),
        ],
        out_specs=[
            pl.BlockSpec((_BM, 128), row), pl.BlockSpec((_BM, 64), row),
            pl.BlockSpec((_BM, 64), row), pl.BlockSpec((_BM, 64), row),
        ],
        out_shape=[
            jax.ShapeDtypeStruct((_NPAD, 128), _F32),
            jax.ShapeDtypeStruct((_NPAD, 64), _F32),
            jax.ShapeDtypeStruct((_NPAD, 64), _F32),
            jax.ShapeDtypeStruct((_NPAD, 64), _F32),
        ],
    )(pA[0], pA[1], pB[0], pB[1], b_r1, W_r2, b_r2, W_f0, b_f0, b_g0, W_g1)


def _tc3_body(pC0, pC1, self_r, dinv_r, bg1, wd1, bd1, wd2, bd2, H2_o, X_o):
    S2 = pC0[...] + pC1[...]
    H2 = jnp.maximum(dinv_r[...] * S2 + self_r[...] + bg1[...], 0.0)
    T = jnp.maximum(
        jnp.dot(H2, wd1[...], preferred_element_type=_F32) + bd1[...], 0.0)
    X = jnp.maximum(
        jnp.dot(T, wd2[...], preferred_element_type=_F32) + bd2[...], 0.0)
    H2_o[...] = H2
    X_o[...] = X


def _tc_dec(pC, selfterm, dinv64, b_g1, W_d1, b_d1, W_d2, b_d2):
    g = _NPAD // _BM
    row = lambda i: (i, 0)
    cst = lambda i: (0, 0)
    return pl.pallas_call(
        _tc3_body,
        grid=(g,),
        in_specs=[
            pl.BlockSpec((_BM, 64), row), pl.BlockSpec((_BM, 64), row),
            pl.BlockSpec((_BM, 64), row), pl.BlockSpec((_BM, 64), row),
            pl.BlockSpec((1, 64), cst), pl.BlockSpec((64, 128), cst),
            pl.BlockSpec((1, 128), cst), pl.BlockSpec((128, 128), cst),
            pl.BlockSpec((1, 128), cst),
        ],
        out_specs=[pl.BlockSpec((_BM, 64), row), pl.BlockSpec((_BM, 128), row)],
        out_shape=[jax.ShapeDtypeStruct((_NPAD, 64), _F32),
                   jax.ShapeDtypeStruct((_NPAD, 128), _F32)],
    )(pC[0], pC[1], selfterm, dinv64, b_g1, W_d1, b_d1, W_d2, b_d2)


def _tc4_body(a_ref, b_ref, o_ref):
    o_ref[...] = lax.dot_general(
        a_ref[...], b_ref[...], (((1,), (1,)), ((), ())),
        preferred_element_type=_F32)


def _tc_ahat(H2):
    BM = 400
    return pl.pallas_call(
        _tc4_body,
        grid=(_N // BM,),
        in_specs=[pl.BlockSpec((BM, 64), lambda i: (i, 0)),
                  pl.BlockSpec((_N, 64), lambda i: (0, 0))],
        out_specs=pl.BlockSpec((BM, _N), lambda i: (i, 0)),
        out_shape=jax.ShapeDtypeStruct((_N, _N), _F32),
    )(H2, H2)


def kernel(x, W_r1, b_r1, W_r2, b_r2, W_f0, b_f0, W_g0, b_g0, W_g1, b_g1,
           W_d1, b_d1, W_d2, b_d2, edge_index):
    src = edge_index[0].astype(jnp.int32)
    dst = edge_index[1].astype(jnp.int32)
    pad = jnp.full((_EPAD - _E,), _N, jnp.int32)
    srcp = jnp.concatenate([src, pad])
    dstp = jnp.concatenate([dst, pad])

    xp = jnp.pad(x, ((0, _NPAD - _N), (0, 0)))
    W_r1p = jnp.pad(W_r1, ((0, _NPAD - _N), (0, 0)))

    r2 = lambda b: b.reshape(1, -1)

    m0ext = _tc_m0ext(xp, W_g0)
    pA = _sc_edge_scatter(W_r1p, dstp, srcp, 128, 128, 2)
    pB = _sc_edge_scatter(m0ext, srcp, dstp, 144, 80, 2)
    R, m1s, selfterm, dinv64 = _tc_mid(
        pA, pB, r2(b_r1), W_r2, r2(b_r2), W_f0, r2(b_f0), r2(b_g0), W_g1)
    pC = _sc_edge_scatter(m1s, srcp, dstp, 64, 128, 2)
    H2, X_hat = _tc_dec(pC, selfterm, dinv64, r2(b_g1), W_d1, r2(b_d1),
                        W_d2, r2(b_d2))
    A_hat = _tc_ahat(H2[:_N])
    return X_hat[:_N], A_hat, R[:_N]


# speedup vs baseline: 1.4146x; 1.2996x over previous
import functools

import jax
import jax.numpy as jnp
from jax import lax
from jax.experimental import pallas as pl
from jax.experimental.pallas import tpu as pltpu
from jax.experimental.pallas import tpu_sc as plsc

_N = 10000
_NPAD = 10112
_RPT = _NPAD // 16
_E = 160000
_CORES = 2
_TILES = 16
_EPT = 5120
_EPAD = _CORES * _TILES * _EPT
_GAMMA = 0.5
_F32 = jnp.float32


def _sc_edge_scatter(table, gidx, sidx, width, g, nbuf):
    chunks = _EPT // g
    assert _EPT % g == 0 and chunks % nbuf == 0
    dt = table.dtype
    gidx = gidx.reshape(_CORES, _TILES, chunks, g)
    sidx = sidx.reshape(_CORES, _TILES, chunks, g)
    mesh = plsc.VectorSubcoreMesh(core_axis_name="c", subcore_axis_name="s")
    zeros = jnp.zeros((_RPT, width), dt)

    def body(table_h, gidx_h, sidx_h, zeros_h, out_h, gi_v, si_v,
             gbufs, acc_s, sems_g, sems_s):
        c = lax.axis_index("c")
        s = lax.axis_index("s")
        r0 = s * _RPT
        pltpu.sync_copy(zeros_h, acc_s.at[pl.ds(r0, _RPT)])
        pltpu.sync_copy(gidx_h.at[c, s], gi_v)
        pltpu.sync_copy(sidx_h.at[c, s], si_v)
        plsc.subcore_barrier()

        for b in range(nbuf):
            pltpu.async_copy(table_h.at[gi_v.at[b]], gbufs[b], sems_g[b])

        last = chunks - 1

        @pl.loop(0, chunks, step=nbuf)
        def _chunk(j):
            for b in range(nbuf):
                pltpu.make_async_copy(table_h.at[gi_v.at[0]], gbufs[b],
                                      sems_g[b]).wait()
                pltpu.async_copy(gbufs[b], acc_s.at[si_v.at[j + b]],
                                 sems_s[b], add=True).wait()
                pltpu.async_copy(
                    table_h.at[gi_v.at[jnp.minimum(j + nbuf + b, last)]],
                    gbufs[b], sems_g[b])

        for b in range(nbuf):
            pltpu.make_async_copy(table_h.at[gi_v.at[0]], gbufs[b],
                                  sems_g[b]).wait()
        plsc.subcore_barrier()
        pltpu.sync_copy(acc_s.at[pl.ds(r0, _RPT)],
                        out_h.at[c, pl.ds(r0, _RPT)])

    fn = pl.kernel(
        body,
        out_type=jax.ShapeDtypeStruct((_CORES, _NPAD, width), dt),
        mesh=mesh,
        scratch_types=[
            pltpu.VMEM((chunks, g), jnp.int32),
            pltpu.VMEM((chunks, g), jnp.int32),
            [pltpu.VMEM((g, width), dt) for _ in range(nbuf)],
            pltpu.VMEM_SHARED((_NPAD, width), dt),
            [pltpu.SemaphoreType.DMA for _ in range(nbuf)],
            [pltpu.SemaphoreType.DMA for _ in range(nbuf)],
        ],
        compiler_params=pltpu.CompilerParams(use_tc_tiling_on_sc=False),
    )
    return fn(table, gidx, sidx, zeros)


_BM = 1264


def _tc1_body(x_ref, w_ref, o_ref):
    i = pl.program_id(0)
    m = jnp.dot(x_ref[...], w_ref[...], preferred_element_type=_F32)
    row = i * _BM + lax.broadcasted_iota(jnp.int32, (_BM, 32), 0)
    col = lax.broadcasted_iota(jnp.int32, (_BM, 32), 1)
    ones = jnp.where((row < _N) & (col == 0), 1.0, 0.0).astype(_F32)
    o_ref[...] = jnp.concatenate([m, ones], axis=1).astype(jnp.bfloat16)


def _tc_m0ext(x_pad, W_g0):
    return pl.pallas_call(
        _tc1_body,
        grid=(_NPAD // _BM,),
        in_specs=[pl.BlockSpec((_BM, 128), lambda i: (i, 0)),
                  pl.BlockSpec((128, 128), lambda i: (0, 0))],
        out_specs=pl.BlockSpec((_BM, 160), lambda i: (i, 0)),
        out_shape=jax.ShapeDtypeStruct((_NPAD, 160), jnp.bfloat16),
    )(x_pad, W_g0)


def _tc2_body(pA0, pA1, pB0, pB1, br1, wr2, br2, wf0, bf0, bg0, wg1,
              R_o, m1s_o, self_o, dinv_o):
    AW = pA0[...].astype(_F32) + pA1[...].astype(_F32)
    T1 = jnp.maximum(AW + br1[...], 0.0)
    R = jnp.maximum(
        jnp.dot(T1, wr2[...], preferred_element_type=_F32) + br2[...], 0.0)
    Rl = jnp.maximum(
        jnp.dot(R, wf0[...], preferred_element_type=_F32) + bf0[...], 0.0)
    S1e = pB0[...].astype(_F32) + pB1[...].astype(_F32)
    H = jnp.maximum(S1e[:, :128] + bg0[...], 0.0)
    Hm = H * jnp.exp(-_GAMMA * Rl)
    m1 = jnp.dot(Hm, wg1[...], preferred_element_type=_F32)
    deg = S1e[:, 128:129] + 1.0
    dinv = lax.rsqrt(deg)
    dinv64 = jnp.broadcast_to(dinv, (_BM, 64))
    R_o[...] = R
    m1s_o[...] = (m1 * dinv64).astype(jnp.bfloat16)
    self_o[...] = m1 * dinv64 * dinv64
    dinv_o[...] = dinv64


def _tc_mid(pA, pB, b_r1, W_r2, b_r2, W_f0, b_f0, b_g0, W_g1):
    g = _NPAD // _BM
    row = lambda i: (i, 0)
    cst = lambda i: (0, 0)
    return pl.pallas_call(
        _tc2_body,
        grid=(g,),
        in_specs=[
            pl.BlockSpec((_BM, 128), row), pl.BlockSpec((_BM, 128), row),
            pl.BlockSpec((_BM, 160), row), pl.BlockSpec((_BM, 160), row),
            pl.BlockSpec((1, 128), cst), pl.BlockSpec((128, 128), cst),
            pl.BlockSpec((1, 128), cst), pl.BlockSpec((128, 128), cst),
            pl.BlockSpec((1, 128), cst), pl.BlockSpec((1, 128), cst),
            pl.BlockSpec((128, 64), cst),
        ],
        out_specs=[
            pl.BlockSpec((_BM, 128), row), pl.BlockSpec((_BM, 64), row),
            pl.BlockSpec((_BM, 64), row), pl.BlockSpec((_BM, 64), row),
        ],
        out_shape=[
            jax.ShapeDtypeStruct((_NPAD, 128), _F32),
            jax.ShapeDtypeStruct((_NPAD, 64), jnp.bfloat16),
            jax.ShapeDtypeStruct((_NPAD, 64), _F32),
            jax.ShapeDtypeStruct((_NPAD, 64), _F32),
        ],
    )(pA[0], pA[1], pB[0], pB[1], b_r1, W_r2, b_r2, W_f0, b_f0, b_g0, W_g1)


def _tc3_body(pC0, pC1, self_r, dinv_r, bg1, wd1, bd1, wd2, bd2, H2_o, X_o):
    S2 = pC0[...].astype(_F32) + pC1[...].astype(_F32)
    H2 = jnp.maximum(dinv_r[...] * S2 + self_r[...] + bg1[...], 0.0)
    T = jnp.maximum(
        jnp.dot(H2, wd1[...], preferred_element_type=_F32) + bd1[...], 0.0)
    X = jnp.maximum(
        jnp.dot(T, wd2[...], preferred_element_type=_F32) + bd2[...], 0.0)
    H2_o[...] = H2
    X_o[...] = X


def _tc_dec(pC, selfterm, dinv64, b_g1, W_d1, b_d1, W_d2, b_d2):
    g = _NPAD // _BM
    row = lambda i: (i, 0)
    cst = lambda i: (0, 0)
    return pl.pallas_call(
        _tc3_body,
        grid=(g,),
        in_specs=[
            pl.BlockSpec((_BM, 64), row), pl.BlockSpec((_BM, 64), row),
            pl.BlockSpec((_BM, 64), row), pl.BlockSpec((_BM, 64), row),
            pl.BlockSpec((1, 64), cst), pl.BlockSpec((64, 128), cst),
            pl.BlockSpec((1, 128), cst), pl.BlockSpec((128, 128), cst),
            pl.BlockSpec((1, 128), cst),
        ],
        out_specs=[pl.BlockSpec((_BM, 64), row), pl.BlockSpec((_BM, 128), row)],
        out_shape=[jax.ShapeDtypeStruct((_NPAD, 64), _F32),
                   jax.ShapeDtypeStruct((_NPAD, 128), _F32)],
    )(pC[0], pC[1], selfterm, dinv64, b_g1, W_d1, b_d1, W_d2, b_d2)


def _tc4_body(a_ref, b_ref, o_ref):
    o_ref[...] = lax.dot_general(
        a_ref[...], b_ref[...], (((1,), (1,)), ((), ())),
        preferred_element_type=_F32)


def _tc_ahat(H2):
    BM = 400
    return pl.pallas_call(
        _tc4_body,
        grid=(_N // BM,),
        in_specs=[pl.BlockSpec((BM, 64), lambda i: (i, 0)),
                  pl.BlockSpec((_N, 64), lambda i: (0, 0))],
        out_specs=pl.BlockSpec((BM, _N), lambda i: (i, 0)),
        out_shape=jax.ShapeDtypeStruct((_N, _N), _F32),
    )(H2, H2)


def kernel(x, W_r1, b_r1, W_r2, b_r2, W_f0, b_f0, W_g0, b_g0, W_g1, b_g1,
           W_d1, b_d1, W_d2, b_d2, edge_index):
    src = edge_index[0].astype(jnp.int32)
    dst = edge_index[1].astype(jnp.int32)
    pad = jnp.full((_EPAD - _E,), _N, jnp.int32)
    srcp = jnp.concatenate([src, pad])
    dstp = jnp.concatenate([dst, pad])

    xp = jnp.pad(x, ((0, _NPAD - _N), (0, 0)))
    W_r1p = jnp.pad(W_r1, ((0, _NPAD - _N), (0, 0))).astype(jnp.bfloat16)

    r2 = lambda b: b.reshape(1, -1)

    m0ext = _tc_m0ext(xp, W_g0)
    pA = _sc_edge_scatter(W_r1p, dstp, srcp, 128, 128, 2)
    pB = _sc_edge_scatter(m0ext, srcp, dstp, 160, 128, 2)
    R, m1s, selfterm, dinv64 = _tc_mid(
        pA, pB, r2(b_r1), W_r2, r2(b_r2), W_f0, r2(b_f0), r2(b_g0), W_g1)
    pC = _sc_edge_scatter(m1s, srcp, dstp, 64, 128, 2)
    H2, X_hat = _tc_dec(pC, selfterm, dinv64, r2(b_g1), W_d1, r2(b_d1),
                        W_d2, r2(b_d2))
    A_hat = _tc_ahat(H2[:_N])
    return X_hat[:_N], A_hat, R[:_N]


# speedup vs baseline: 1.4695x; 1.0388x over previous
import functools

import jax
import jax.numpy as jnp
from jax import lax
from jax.experimental import pallas as pl
from jax.experimental.pallas import tpu as pltpu
from jax.experimental.pallas import tpu_sc as plsc

_N = 10000
_NPAD = 10112
_RPT = _NPAD // 16
_E = 160000
_CORES = 2
_TILES = 16
_EPT = 5120
_EPAD = _CORES * _TILES * _EPT
_GAMMA = 0.5
_F32 = jnp.float32


def _sc_edge_scatter(table, gidx, sidx, width, g, nbuf):
    chunks = _EPT // g
    assert _EPT % g == 0 and chunks % nbuf == 0
    dt = table.dtype
    gidx = gidx.reshape(_CORES, _TILES, chunks, g)
    sidx = sidx.reshape(_CORES, _TILES, chunks, g)
    mesh = plsc.VectorSubcoreMesh(core_axis_name="c", subcore_axis_name="s")
    zeros = jnp.zeros((_RPT, width), dt)

    def body(table_h, gidx_h, sidx_h, zeros_h, out_h, gi_v, si_v,
             gbufs, acc_s, sems_g, sems_s):
        c = lax.axis_index("c")
        s = lax.axis_index("s")
        r0 = s * _RPT
        pltpu.sync_copy(zeros_h, acc_s.at[pl.ds(r0, _RPT)])
        pltpu.sync_copy(gidx_h.at[c, s], gi_v)
        pltpu.sync_copy(sidx_h.at[c, s], si_v)
        plsc.subcore_barrier()

        for b in range(nbuf):
            pltpu.async_copy(table_h.at[gi_v.at[b]], gbufs[b], sems_g[b])

        last = chunks - 1

        @pl.loop(0, chunks, step=nbuf)
        def _chunk(j):
            for b in range(nbuf):
                pltpu.make_async_copy(table_h.at[gi_v.at[0]], gbufs[b],
                                      sems_g[b]).wait()
                pltpu.async_copy(gbufs[b], acc_s.at[si_v.at[j + b]],
                                 sems_s[b], add=True).wait()
                pltpu.async_copy(
                    table_h.at[gi_v.at[jnp.minimum(j + nbuf + b, last)]],
                    gbufs[b], sems_g[b])

        for b in range(nbuf):
            pltpu.make_async_copy(table_h.at[gi_v.at[0]], gbufs[b],
                                  sems_g[b]).wait()
        plsc.subcore_barrier()
        pltpu.sync_copy(acc_s.at[pl.ds(r0, _RPT)],
                        out_h.at[c, pl.ds(r0, _RPT)])

    fn = pl.kernel(
        body,
        out_type=jax.ShapeDtypeStruct((_CORES, _NPAD, width), dt),
        mesh=mesh,
        scratch_types=[
            pltpu.VMEM((chunks, g), jnp.int32),
            pltpu.VMEM((chunks, g), jnp.int32),
            [pltpu.VMEM((g, width), dt) for _ in range(nbuf)],
            pltpu.VMEM_SHARED((_NPAD, width), dt),
            [pltpu.SemaphoreType.DMA for _ in range(nbuf)],
            [pltpu.SemaphoreType.DMA for _ in range(nbuf)],
        ],
        compiler_params=pltpu.CompilerParams(use_tc_tiling_on_sc=False),
    )
    return fn(table, gidx, sidx, zeros)


_G = 128
_CHUNKS = _EPT // _G


def _sc_ab_deg(tabA, tabB, srcp, dstp):
    src_r = srcp.reshape(_CORES, _TILES, _CHUNKS, _G)
    dst_r = dstp.reshape(_CORES, _TILES, _CHUNKS, _G)
    mesh = plsc.VectorSubcoreMesh(core_axis_name="c", subcore_axis_name="s")
    bf = jnp.bfloat16
    zerosA = jnp.zeros((_RPT, 128), bf)
    zerosD = jnp.zeros((_RPT, 16), _F32)
    onesD = jnp.concatenate(
        [jnp.ones((_G, 1), _F32), jnp.zeros((_G, 15), _F32)], axis=1)

    def ring(table_h, gi_v, si_v, gbufs, acc_s, sems_g, sems_s):
        for b in range(2):
            pltpu.async_copy(table_h.at[gi_v.at[b]], gbufs[b], sems_g[b])
        last = _CHUNKS - 1

        @pl.loop(0, _CHUNKS, step=2)
        def _chunk(j):
            for b in range(2):
                pltpu.make_async_copy(table_h.at[gi_v.at[0]], gbufs[b],
                                      sems_g[b]).wait()
                pltpu.async_copy(gbufs[b], acc_s.at[si_v.at[j + b]],
                                 sems_s[b], add=True).wait()
                pltpu.async_copy(
                    table_h.at[gi_v.at[jnp.minimum(j + 2 + b, last)]],
                    gbufs[b], sems_g[b])

        for b in range(2):
            pltpu.make_async_copy(table_h.at[gi_v.at[0]], gbufs[b],
                                  sems_g[b]).wait()

    def body(tabA_h, tabB_h, src_h, dst_h, zerosA_h, zerosD_h, onesD_h,
             outA_h, outB_h, outD_h,
             si_v, di_v, ones_v, gbufs, accA_s, accB_s, accD_s,
             sems_g, sems_s):
        c = lax.axis_index("c")
        s = lax.axis_index("s")
        r0 = s * _RPT
        pltpu.sync_copy(zerosA_h, accA_s.at[pl.ds(r0, _RPT)])
        pltpu.sync_copy(zerosA_h, accB_s.at[pl.ds(r0, _RPT)])
        pltpu.sync_copy(zerosD_h, accD_s.at[pl.ds(r0, _RPT)])
        pltpu.sync_copy(src_h.at[c, s], si_v)
        pltpu.sync_copy(dst_h.at[c, s], di_v)
        pltpu.sync_copy(onesD_h, ones_v)
        plsc.subcore_barrier()

        ring(tabA_h, di_v, si_v, gbufs, accA_s, sems_g, sems_s)
        ring(tabB_h, si_v, di_v, gbufs, accB_s, sems_g, sems_s)

        for b in range(2):
            pltpu.async_copy(ones_v, accD_s.at[di_v.at[b]], sems_s[b],
                             add=True)

        @pl.loop(0, _CHUNKS - 2, step=2)
        def _dchunk(j):
            for b in range(2):
                pltpu.make_async_copy(ones_v, accD_s.at[di_v.at[0]],
                                      sems_s[b]).wait()
                pltpu.async_copy(ones_v, accD_s.at[di_v.at[j + 2 + b]],
                                 sems_s[b], add=True)

        for b in range(2):
            pltpu.make_async_copy(ones_v, accD_s.at[di_v.at[0]],
                                  sems_s[b]).wait()
        plsc.subcore_barrier()
        pltpu.sync_copy(accA_s.at[pl.ds(r0, _RPT)],
                        outA_h.at[c, pl.ds(r0, _RPT)])
        pltpu.sync_copy(accB_s.at[pl.ds(r0, _RPT)],
                        outB_h.at[c, pl.ds(r0, _RPT)])
        pltpu.sync_copy(accD_s.at[pl.ds(r0, _RPT)],
                        outD_h.at[c, pl.ds(r0, _RPT)])

    fn = pl.kernel(
        body,
        out_type=(jax.ShapeDtypeStruct((_CORES, _NPAD, 128), bf),
                  jax.ShapeDtypeStruct((_CORES, _NPAD, 128), bf),
                  jax.ShapeDtypeStruct((_CORES, _NPAD, 16), _F32)),
        mesh=mesh,
        scratch_types=[
            pltpu.VMEM((_CHUNKS, _G), jnp.int32),
            pltpu.VMEM((_CHUNKS, _G), jnp.int32),
            pltpu.VMEM((_G, 16), _F32),
            [pltpu.VMEM((_G, 128), bf) for _ in range(2)],
            pltpu.VMEM_SHARED((_NPAD, 128), bf),
            pltpu.VMEM_SHARED((_NPAD, 128), bf),
            pltpu.VMEM_SHARED((_NPAD, 16), _F32),
            [pltpu.SemaphoreType.DMA for _ in range(2)],
            [pltpu.SemaphoreType.DMA for _ in range(2)],
        ],
        compiler_params=pltpu.CompilerParams(use_tc_tiling_on_sc=False),
    )
    return fn(tabA, tabB, src_r, dst_r, zerosA, zerosD, onesD)


_BM = 1264


def _tc1_body(x_ref, w_ref, o_ref):
    m = jnp.dot(x_ref[...], w_ref[...], preferred_element_type=_F32)
    o_ref[...] = m.astype(jnp.bfloat16)


def _tc_m0ext(x_pad, W_g0):
    return pl.pallas_call(
        _tc1_body,
        grid=(_NPAD // _BM,),
        in_specs=[pl.BlockSpec((_BM, 128), lambda i: (i, 0)),
                  pl.BlockSpec((128, 128), lambda i: (0, 0))],
        out_specs=pl.BlockSpec((_BM, 128), lambda i: (i, 0)),
        out_shape=jax.ShapeDtypeStruct((_NPAD, 128), jnp.bfloat16),
    )(x_pad, W_g0)


def _tc2_body(pA0, pA1, pB0, pB1, pD0, pD1, br1, wr2, br2, wf0, bf0, bg0,
              wg1, R_o, m1s_o, self_o, dinv_o):
    AW = pA0[...].astype(_F32) + pA1[...].astype(_F32)
    T1 = jnp.maximum(AW + br1[...], 0.0)
    R = jnp.maximum(
        jnp.dot(T1, wr2[...], preferred_element_type=_F32) + br2[...], 0.0)
    Rl = jnp.maximum(
        jnp.dot(R, wf0[...], preferred_element_type=_F32) + bf0[...], 0.0)
    S1 = pB0[...].astype(_F32) + pB1[...].astype(_F32)
    H = jnp.maximum(S1 + bg0[...], 0.0)
    Hm = H * jnp.exp(-_GAMMA * Rl)
    m1 = jnp.dot(Hm, wg1[...], preferred_element_type=_F32)
    deg = pD0[:, 0:1] + pD1[:, 0:1] + 1.0
    dinv = lax.rsqrt(deg)
    dinv64 = jnp.broadcast_to(dinv, (_BM, 64))
    R_o[...] = R
    m1s_o[...] = (m1 * dinv64).astype(jnp.bfloat16)
    self_o[...] = m1 * dinv64 * dinv64
    dinv_o[...] = dinv64


def _tc_mid(pA, pB, pD, b_r1, W_r2, b_r2, W_f0, b_f0, b_g0, W_g1):
    g = _NPAD // _BM
    row = lambda i: (i, 0)
    cst = lambda i: (0, 0)
    return pl.pallas_call(
        _tc2_body,
        grid=(g,),
        in_specs=[
            pl.BlockSpec((_BM, 128), row), pl.BlockSpec((_BM, 128), row),
            pl.BlockSpec((_BM, 128), row), pl.BlockSpec((_BM, 128), row),
            pl.BlockSpec((_BM, 16), row), pl.BlockSpec((_BM, 16), row),
            pl.BlockSpec((1, 128), cst), pl.BlockSpec((128, 128), cst),
            pl.BlockSpec((1, 128), cst), pl.BlockSpec((128, 128), cst),
            pl.BlockSpec((1, 128), cst), pl.BlockSpec((1, 128), cst),
            pl.BlockSpec((128, 64), cst),
        ],
        out_specs=[
            pl.BlockSpec((_BM, 128), row), pl.BlockSpec((_BM, 64), row),
            pl.BlockSpec((_BM, 64), row), pl.BlockSpec((_BM, 64), row),
        ],
        out_shape=[
            jax.ShapeDtypeStruct((_NPAD, 128), _F32),
            jax.ShapeDtypeStruct((_NPAD, 64), jnp.bfloat16),
            jax.ShapeDtypeStruct((_NPAD, 64), _F32),
            jax.ShapeDtypeStruct((_NPAD, 64), _F32),
        ],
    )(pA[0], pA[1], pB[0], pB[1], pD[0], pD[1], b_r1, W_r2, b_r2, W_f0,
      b_f0, b_g0, W_g1)


def _tc3_body(pC0, pC1, self_r, dinv_r, bg1, wd1, bd1, wd2, bd2, H2_o, X_o):
    S2 = pC0[...].astype(_F32) + pC1[...].astype(_F32)
    H2 = jnp.maximum(dinv_r[...] * S2 + self_r[...] + bg1[...], 0.0)
    T = jnp.maximum(
        jnp.dot(H2, wd1[...], preferred_element_type=_F32) + bd1[...], 0.0)
    X = jnp.maximum(
        jnp.dot(T, wd2[...], preferred_element_type=_F32) + bd2[...], 0.0)
    H2_o[...] = H2
    X_o[...] = X


def _tc_dec(pC, selfterm, dinv64, b_g1, W_d1, b_d1, W_d2, b_d2):
    g = _NPAD // _BM
    row = lambda i: (i, 0)
    cst = lambda i: (0, 0)
    return pl.pallas_call(
        _tc3_body,
        grid=(g,),
        in_specs=[
            pl.BlockSpec((_BM, 64), row), pl.BlockSpec((_BM, 64), row),
            pl.BlockSpec((_BM, 64), row), pl.BlockSpec((_BM, 64), row),
            pl.BlockSpec((1, 64), cst), pl.BlockSpec((64, 128), cst),
            pl.BlockSpec((1, 128), cst), pl.BlockSpec((128, 128), cst),
            pl.BlockSpec((1, 128), cst),
        ],
        out_specs=[pl.BlockSpec((_BM, 64), row), pl.BlockSpec((_BM, 128), row)],
        out_shape=[jax.ShapeDtypeStruct((_NPAD, 64), _F32),
                   jax.ShapeDtypeStruct((_NPAD, 128), _F32)],
    )(pC[0], pC[1], selfterm, dinv64, b_g1, W_d1, b_d1, W_d2, b_d2)


def _tc4_body(a_ref, b_ref, o_ref):
    o_ref[...] = lax.dot_general(
        a_ref[...], b_ref[...], (((1,), (1,)), ((), ())),
        preferred_element_type=_F32)


def _tc_ahat(H2):
    BM = 400
    return pl.pallas_call(
        _tc4_body,
        grid=(_N // BM,),
        in_specs=[pl.BlockSpec((BM, 64), lambda i: (i, 0)),
                  pl.BlockSpec((_N, 64), lambda i: (0, 0))],
        out_specs=pl.BlockSpec((BM, _N), lambda i: (i, 0)),
        out_shape=jax.ShapeDtypeStruct((_N, _N), _F32),
    )(H2, H2)


def kernel(x, W_r1, b_r1, W_r2, b_r2, W_f0, b_f0, W_g0, b_g0, W_g1, b_g1,
           W_d1, b_d1, W_d2, b_d2, edge_index):
    src = edge_index[0].astype(jnp.int32)
    dst = edge_index[1].astype(jnp.int32)
    pad = jnp.full((_EPAD - _E,), _N, jnp.int32)
    srcp = jnp.concatenate([src, pad])
    dstp = jnp.concatenate([dst, pad])

    xp = jnp.pad(x, ((0, _NPAD - _N), (0, 0)))
    W_r1p = jnp.pad(W_r1, ((0, _NPAD - _N), (0, 0))).astype(jnp.bfloat16)

    r2 = lambda b: b.reshape(1, -1)

    m0 = _tc_m0ext(xp, W_g0)
    pA, pB, pD = _sc_ab_deg(W_r1p, m0, srcp, dstp)
    R, m1s, selfterm, dinv64 = _tc_mid(
        pA, pB, pD, r2(b_r1), W_r2, r2(b_r2), W_f0, r2(b_f0), r2(b_g0), W_g1)
    pC = _sc_edge_scatter(m1s, srcp, dstp, 64, 128, 2)
    H2, X_hat = _tc_dec(pC, selfterm, dinv64, r2(b_g1), W_d1, r2(b_d1),
                        W_d2, r2(b_d2))
    A_hat = _tc_ahat(H2[:_N])
    return X_hat[:_N], A_hat, R[:_N]


# speedup vs baseline: 1.4700x; 1.0003x over previous
import functools

import jax
import jax.numpy as jnp
from jax import lax
from jax.experimental import pallas as pl
from jax.experimental.pallas import tpu as pltpu
from jax.experimental.pallas import tpu_sc as plsc

_N = 10000
_NPAD = 10112
_RPT = _NPAD // 16
_E = 160000
_CORES = 2
_TILES = 16
_EPT = 5120
_EPAD = _CORES * _TILES * _EPT
_GAMMA = 0.5
_F32 = jnp.float32


def _sc_edge_scatter(table, gidx, sidx, width, g, nbuf):
    chunks = _EPT // g
    assert _EPT % g == 0 and chunks % nbuf == 0
    dt = table.dtype
    gidx = gidx.reshape(_CORES, _TILES, chunks, g)
    sidx = sidx.reshape(_CORES, _TILES, chunks, g)
    mesh = plsc.VectorSubcoreMesh(core_axis_name="c", subcore_axis_name="s")
    zeros = jnp.zeros((_RPT, width), dt)

    def body(table_h, gidx_h, sidx_h, zeros_h, out_h, gi_v, si_v,
             gbufs, acc_s, sems_g, sems_s):
        c = lax.axis_index("c")
        s = lax.axis_index("s")
        r0 = s * _RPT
        pltpu.sync_copy(zeros_h, acc_s.at[pl.ds(r0, _RPT)])
        pltpu.sync_copy(gidx_h.at[c, s], gi_v)
        pltpu.sync_copy(sidx_h.at[c, s], si_v)
        plsc.subcore_barrier()

        for b in range(nbuf):
            pltpu.async_copy(table_h.at[gi_v.at[b]], gbufs[b], sems_g[b])

        last = chunks - 1

        @pl.loop(0, chunks, step=nbuf)
        def _chunk(j):
            for b in range(nbuf):
                pltpu.make_async_copy(table_h.at[gi_v.at[0]], gbufs[b],
                                      sems_g[b]).wait()
                pltpu.async_copy(gbufs[b], acc_s.at[si_v.at[j + b]],
                                 sems_s[b], add=True).wait()
                pltpu.async_copy(
                    table_h.at[gi_v.at[jnp.minimum(j + nbuf + b, last)]],
                    gbufs[b], sems_g[b])

        for b in range(nbuf):
            pltpu.make_async_copy(table_h.at[gi_v.at[0]], gbufs[b],
                                  sems_g[b]).wait()
        plsc.subcore_barrier()
        pltpu.sync_copy(acc_s.at[pl.ds(r0, _RPT)],
                        out_h.at[c, pl.ds(r0, _RPT)])

    fn = pl.kernel(
        body,
        out_type=jax.ShapeDtypeStruct((_CORES, _NPAD, width), dt),
        mesh=mesh,
        scratch_types=[
            pltpu.VMEM((chunks, g), jnp.int32),
            pltpu.VMEM((chunks, g), jnp.int32),
            [pltpu.VMEM((g, width), dt) for _ in range(nbuf)],
            pltpu.VMEM_SHARED((_NPAD, width), dt),
            [pltpu.SemaphoreType.DMA for _ in range(nbuf)],
            [pltpu.SemaphoreType.DMA for _ in range(nbuf)],
        ],
        compiler_params=pltpu.CompilerParams(use_tc_tiling_on_sc=False),
    )
    return fn(table, gidx, sidx, zeros)


_G = 128
_CHUNKS = _EPT // _G


def _sc_ab_deg(tabA, tabB, srcp, dstp):
    src_r = srcp.reshape(_CORES, _TILES, _CHUNKS, _G)
    dst_r = dstp.reshape(_CORES, _TILES, _CHUNKS, _G)
    mesh = plsc.VectorSubcoreMesh(core_axis_name="c", subcore_axis_name="s")
    bf = jnp.bfloat16
    zerosA = jnp.zeros((_RPT, 128), bf)
    zerosD = jnp.zeros((_RPT, 16), _F32)
    onesD = jnp.concatenate(
        [jnp.ones((_G, 1), _F32), jnp.zeros((_G, 15), _F32)], axis=1)

    def ring(table_h, gi_v, si_v, gbufs, acc_s, sems_g, sems_s):
        for b in range(2):
            pltpu.async_copy(table_h.at[gi_v.at[b]], gbufs[b], sems_g[b])
        last = _CHUNKS - 1

        @pl.loop(0, _CHUNKS, step=2)
        def _chunk(j):
            for b in range(2):
                pltpu.make_async_copy(table_h.at[gi_v.at[0]], gbufs[b],
                                      sems_g[b]).wait()
                pltpu.async_copy(gbufs[b], acc_s.at[si_v.at[j + b]],
                                 sems_s[b], add=True).wait()
                pltpu.async_copy(
                    table_h.at[gi_v.at[jnp.minimum(j + 2 + b, last)]],
                    gbufs[b], sems_g[b])

        for b in range(2):
            pltpu.make_async_copy(table_h.at[gi_v.at[0]], gbufs[b],
                                  sems_g[b]).wait()

    def body(tabA_h, tabB_h, src_h, dst_h, zerosA_h, zerosD_h, onesD_h,
             outA_h, outB_h, outD_h,
             si_v, di_v, ones_v, gbufs, accA_s, accB_s, accD_s,
             sems_g, sems_s):
        c = lax.axis_index("c")
        s = lax.axis_index("s")
        r0 = s * _RPT
        pltpu.sync_copy(zerosA_h, accA_s.at[pl.ds(r0, _RPT)])
        pltpu.sync_copy(zerosA_h, accB_s.at[pl.ds(r0, _RPT)])
        pltpu.sync_copy(zerosD_h, accD_s.at[pl.ds(r0, _RPT)])
        pltpu.sync_copy(src_h.at[c, s], si_v)
        pltpu.sync_copy(dst_h.at[c, s], di_v)
        pltpu.sync_copy(onesD_h, ones_v)
        plsc.subcore_barrier()

        ring(tabA_h, di_v, si_v, gbufs, accA_s, sems_g, sems_s)
        ring(tabB_h, si_v, di_v, gbufs, accB_s, sems_g, sems_s)

        for b in range(2):
            pltpu.async_copy(ones_v, accD_s.at[di_v.at[b]], sems_s[b],
                             add=True)

        @pl.loop(0, _CHUNKS - 2, step=2)
        def _dchunk(j):
            for b in range(2):
                pltpu.make_async_copy(ones_v, accD_s.at[di_v.at[0]],
                                      sems_s[b]).wait()
                pltpu.async_copy(ones_v, accD_s.at[di_v.at[j + 2 + b]],
                                 sems_s[b], add=True)

        for b in range(2):
            pltpu.make_async_copy(ones_v, accD_s.at[di_v.at[0]],
                                  sems_s[b]).wait()
        plsc.subcore_barrier()
        pltpu.sync_copy(accA_s.at[pl.ds(r0, _RPT)],
                        outA_h.at[c, pl.ds(r0, _RPT)])
        pltpu.sync_copy(accB_s.at[pl.ds(r0, _RPT)],
                        outB_h.at[c, pl.ds(r0, _RPT)])
        pltpu.sync_copy(accD_s.at[pl.ds(r0, _RPT)],
                        outD_h.at[c, pl.ds(r0, _RPT)])

    fn = pl.kernel(
        body,
        out_type=(jax.ShapeDtypeStruct((_CORES, _NPAD, 128), bf),
                  jax.ShapeDtypeStruct((_CORES, _NPAD, 128), bf),
                  jax.ShapeDtypeStruct((_CORES, _NPAD, 16), _F32)),
        mesh=mesh,
        scratch_types=[
            pltpu.VMEM((_CHUNKS, _G), jnp.int32),
            pltpu.VMEM((_CHUNKS, _G), jnp.int32),
            pltpu.VMEM((_G, 16), _F32),
            [pltpu.VMEM((_G, 128), bf) for _ in range(2)],
            pltpu.VMEM_SHARED((_NPAD, 128), bf),
            pltpu.VMEM_SHARED((_NPAD, 128), bf),
            pltpu.VMEM_SHARED((_NPAD, 16), _F32),
            [pltpu.SemaphoreType.DMA for _ in range(2)],
            [pltpu.SemaphoreType.DMA for _ in range(2)],
        ],
        compiler_params=pltpu.CompilerParams(use_tc_tiling_on_sc=False),
    )
    return fn(tabA, tabB, src_r, dst_r, zerosA, zerosD, onesD)


_BM = 1264


def _tc1_body(x_ref, w_ref, o_ref):
    m = jnp.dot(x_ref[...], w_ref[...], preferred_element_type=_F32)
    o_ref[...] = m.astype(jnp.bfloat16)


def _tc_m0ext(x_pad, W_g0):
    return pl.pallas_call(
        _tc1_body,
        grid=(_NPAD // _BM,),
        in_specs=[pl.BlockSpec((_BM, 128), lambda i: (i, 0)),
                  pl.BlockSpec((128, 128), lambda i: (0, 0))],
        out_specs=pl.BlockSpec((_BM, 128), lambda i: (i, 0)),
        out_shape=jax.ShapeDtypeStruct((_NPAD, 128), jnp.bfloat16),
    )(x_pad, W_g0)


def _tc2_body(pA0, pA1, pB0, pB1, pD0, pD1, br1, wr2, br2, wf0, bf0, bg0,
              wg1, R_o, m1s_o, self_o, dinv_o):
    AW = pA0[...].astype(_F32) + pA1[...].astype(_F32)
    T1 = jnp.maximum(AW + br1[...], 0.0)
    R = jnp.maximum(
        jnp.dot(T1, wr2[...], preferred_element_type=_F32) + br2[...], 0.0)
    Rl = jnp.maximum(
        jnp.dot(R, wf0[...], preferred_element_type=_F32) + bf0[...], 0.0)
    S1 = pB0[...].astype(_F32) + pB1[...].astype(_F32)
    H = jnp.maximum(S1 + bg0[...], 0.0)
    Hm = H * jnp.exp(-_GAMMA * Rl)
    m1 = jnp.dot(Hm, wg1[...], preferred_element_type=_F32)
    deg = pD0[:, 0:1] + pD1[:, 0:1] + 1.0
    dinv = lax.rsqrt(deg)
    dinv64 = jnp.broadcast_to(dinv, (_BM, 64))
    R_o[...] = R
    m1s_o[...] = (m1 * dinv64).astype(jnp.bfloat16)
    self_o[...] = m1 * dinv64 * dinv64
    dinv_o[...] = dinv64


def _tc_mid(pA, pB, pD, b_r1, W_r2, b_r2, W_f0, b_f0, b_g0, W_g1):
    g = _NPAD // _BM
    row = lambda i: (i, 0)
    cst = lambda i: (0, 0)
    return pl.pallas_call(
        _tc2_body,
        grid=(g,),
        in_specs=[
            pl.BlockSpec((_BM, 128), row), pl.BlockSpec((_BM, 128), row),
            pl.BlockSpec((_BM, 128), row), pl.BlockSpec((_BM, 128), row),
            pl.BlockSpec((_BM, 16), row), pl.BlockSpec((_BM, 16), row),
            pl.BlockSpec((1, 128), cst), pl.BlockSpec((128, 128), cst),
            pl.BlockSpec((1, 128), cst), pl.BlockSpec((128, 128), cst),
            pl.BlockSpec((1, 128), cst), pl.BlockSpec((1, 128), cst),
            pl.BlockSpec((128, 64), cst),
        ],
        out_specs=[
            pl.BlockSpec((_BM, 128), row), pl.BlockSpec((_BM, 64), row),
            pl.BlockSpec((_BM, 64), row), pl.BlockSpec((_BM, 64), row),
        ],
        out_shape=[
            jax.ShapeDtypeStruct((_NPAD, 128), _F32),
            jax.ShapeDtypeStruct((_NPAD, 64), jnp.bfloat16),
            jax.ShapeDtypeStruct((_NPAD, 64), _F32),
            jax.ShapeDtypeStruct((_NPAD, 64), _F32),
        ],
    )(pA[0], pA[1], pB[0], pB[1], pD[0], pD[1], b_r1, W_r2, b_r2, W_f0,
      b_f0, b_g0, W_g1)


def _tc3_body(pC0, pC1, self_r, dinv_r, bg1, wd1, bd1, wd2, bd2, H2_o, X_o):
    S2 = pC0[...].astype(_F32) + pC1[...].astype(_F32)
    H2 = jnp.maximum(dinv_r[...] * S2 + self_r[...] + bg1[...], 0.0)
    T = jnp.maximum(
        jnp.dot(H2, wd1[...], preferred_element_type=_F32) + bd1[...], 0.0)
    X = jnp.maximum(
        jnp.dot(T, wd2[...], preferred_element_type=_F32) + bd2[...], 0.0)
    H2_o[...] = H2
    X_o[...] = X


def _tc_dec(pC, selfterm, dinv64, b_g1, W_d1, b_d1, W_d2, b_d2):
    g = _NPAD // _BM
    row = lambda i: (i, 0)
    cst = lambda i: (0, 0)
    return pl.pallas_call(
        _tc3_body,
        grid=(g,),
        in_specs=[
            pl.BlockSpec((_BM, 64), row), pl.BlockSpec((_BM, 64), row),
            pl.BlockSpec((_BM, 64), row), pl.BlockSpec((_BM, 64), row),
            pl.BlockSpec((1, 64), cst), pl.BlockSpec((64, 128), cst),
            pl.BlockSpec((1, 128), cst), pl.BlockSpec((128, 128), cst),
            pl.BlockSpec((1, 128), cst),
        ],
        out_specs=[pl.BlockSpec((_BM, 64), row), pl.BlockSpec((_BM, 128), row)],
        out_shape=[jax.ShapeDtypeStruct((_NPAD, 64), _F32),
                   jax.ShapeDtypeStruct((_NPAD, 128), _F32)],
    )(pC[0], pC[1], selfterm, dinv64, b_g1, W_d1, b_d1, W_d2, b_d2)


def _tc4_body(a_ref, b_ref, o_ref):
    o_ref[...] = lax.dot_general(
        a_ref[...], b_ref[...], (((1,), (1,)), ((), ())),
        preferred_element_type=_F32)


def _tc_ahat(H2):
    BM = 400
    return pl.pallas_call(
        _tc4_body,
        grid=(_N // BM,),
        in_specs=[pl.BlockSpec((BM, 64), lambda i: (i, 0)),
                  pl.BlockSpec((_N, 64), lambda i: (0, 0))],
        out_specs=pl.BlockSpec((BM, _N), lambda i: (i, 0)),
        out_shape=jax.ShapeDtypeStruct((_N, _N), _F32),
    )(H2, H2)


def kernel(x, W_r1, b_r1, W_r2, b_r2, W_f0, b_f0, W_g0, b_g0, W_g1, b_g1,
           W_d1, b_d1, W_d2, b_d2, edge_index):
    src = edge_index[0].astype(jnp.int32)
    dst = edge_index[1].astype(jnp.int32)
    pad = jnp.full((_EPAD - _E,), _N, jnp.int32)
    srcp = jnp.concatenate([src, pad])
    dstp = jnp.concatenate([dst, pad])

    xp = jnp.pad(x, ((0, _NPAD - _N), (0, 0)))
    W_r1p = jnp.pad(W_r1, ((0, _NPAD - _N), (0, 0))).astype(jnp.bfloat16)

    r2 = lambda b: b.reshape(1, -1)

    m0 = _tc_m0ext(xp, W_g0)
    pA, pB, pD = _sc_ab_deg(W_r1p, m0, srcp, dstp)
    R, m1s, selfterm, dinv64 = _tc_mid(
        pA, pB, pD, r2(b_r1), W_r2, r2(b_r2), W_f0, r2(b_f0), r2(b_g0), W_g1)
    pC = _sc_edge_scatter(m1s, srcp, dstp, 64, 128, 4)
    H2, X_hat = _tc_dec(pC, selfterm, dinv64, r2(b_g1), W_d1, r2(b_d1),
                        W_d2, r2(b_d2))
    A_hat = _tc_ahat(H2[:_N])
    return X_hat[:_N], A_hat, R[:_N]
